# speedup vs baseline: 9.1898x; 9.1898x over previous
import functools

import jax
import jax.numpy as jnp
from jax import lax
from jax.experimental import pallas as pl
from jax.experimental.pallas import tpu as pltpu
from jax.experimental.pallas import tpu_sc as plsc

_B = 4096
_L = 200
_D = 128
_N_TOTAL = _B * _L
_NC = 2
_NS = 16
_NW = _NC * _NS
_W = _N_TOTAL // _NW
_G = 128
_K = 4
_BLK = _K * _G
_NBLK = _W // _BLK


@jax.jit
def _gather(idx2d, table):
    mesh = plsc.VectorSubcoreMesh(core_axis_name="c", subcore_axis_name="s")

    @functools.partial(
        pl.kernel,
        mesh=mesh,
        out_type=jax.ShapeDtypeStruct((_N_TOTAL, _D), jnp.float32),
        scratch_types=[
            pltpu.VMEM((_K, _G), jnp.int32),
            pltpu.VMEM((_K, _G, _D), jnp.float32),
            pltpu.SemaphoreType.DMA,
            pltpu.SemaphoreType.DMA,
        ],
    )
    def k(idx_hbm, tab_hbm, out_hbm, idx_v, rows_v, gsem, osem):
        wid = lax.axis_index("s") * _NC + lax.axis_index("c")
        base_row = wid * (_W // _G)

        def blk(i, _):
            row = base_row + i * _K
            pltpu.sync_copy(idx_hbm.at[pl.ds(row, _K)], idx_v)
            for j in range(_K):
                pltpu.async_copy(tab_hbm.at[idx_v.at[j]], rows_v.at[j], gsem)
            for j in range(_K):
                pltpu.make_async_copy(tab_hbm.at[idx_v.at[j]], rows_v.at[j],
                                      gsem).wait()
                pltpu.async_copy(
                    rows_v.at[j],
                    out_hbm.at[pl.ds((row + j) * _G, _G)], osem)
            for j in range(_K):
                pltpu.make_async_copy(
                    rows_v.at[j],
                    out_hbm.at[pl.ds((row + j) * _G, _G)], osem).wait()
            return 0

        lax.fori_loop(0, _NBLK, blk, 0)

    return k(idx2d, table)


def kernel(word_idx, emb0, emb1):
    del emb1
    idx2d = word_idx.reshape(_N_TOTAL // _G, _G)
    out = _gather(idx2d, emb0)
    return out.reshape(_B, _L, _D)


# speedup vs baseline: 9.2170x; 1.0030x over previous
import functools

import jax
import jax.numpy as jnp
from jax import lax
from jax.experimental import pallas as pl
from jax.experimental.pallas import tpu as pltpu
from jax.experimental.pallas import tpu_sc as plsc

_B = 4096
_L = 200
_D = 128
_N_TOTAL = _B * _L
_NC = 2
_NS = 16
_NW = _NC * _NS
_W = _N_TOTAL // _NW
_G = 128
_S = 4
_ROWS = _W // _G
_NOUT = _ROWS // _S


@jax.jit
def _gather(idx2d, table):
    mesh = plsc.VectorSubcoreMesh(core_axis_name="c", subcore_axis_name="s")

    @functools.partial(
        pl.kernel,
        mesh=mesh,
        out_type=jax.ShapeDtypeStruct((_N_TOTAL, _D), jnp.float32),
        scratch_types=[
            pltpu.VMEM((_S, _G), jnp.int32),
            pltpu.VMEM((_S, _G), jnp.int32),
            pltpu.VMEM((_S, _G, _D), jnp.float32),
            pltpu.SemaphoreType.DMA((2,)),
            pltpu.SemaphoreType.DMA((_S,)),
            pltpu.SemaphoreType.DMA((_S,)),
        ],
    )
    def k(idx_hbm, tab_hbm, out_hbm, idx_v0, idx_v1, rows_v, isem, gsem,
          osem):
        idx_bufs = (idx_v0, idx_v1)
        wid = lax.axis_index("s") * _NC + lax.axis_index("c")
        base_row = wid * _ROWS

        pltpu.async_copy(idx_hbm.at[pl.ds(base_row, _S)], idx_v0,
                         isem.at[0])
        pltpu.async_copy(idx_hbm.at[pl.ds(base_row + _S, _S)], idx_v1,
                         isem.at[1])

        def outer(mm, _):
            for p in range(2):
                m5 = 2 * mm + p
                row0 = base_row + m5 * _S
                pltpu.make_async_copy(
                    idx_hbm.at[pl.ds(row0, _S)], idx_bufs[p],
                    isem.at[p]).wait()
                for b in range(_S):
                    if p == 0:
                        @pl.when(mm > 0)
                        def _drain():
                            pltpu.make_async_copy(
                                rows_v.at[b],
                                out_hbm.at[pl.ds((row0 - _S + b) * _G, _G)],
                                osem.at[b]).wait()
                    else:
                        pltpu.make_async_copy(
                            rows_v.at[b],
                            out_hbm.at[pl.ds((row0 - _S + b) * _G, _G)],
                            osem.at[b]).wait()
                    pltpu.async_copy(tab_hbm.at[idx_bufs[p].at[b]],
                                     rows_v.at[b], gsem.at[b])
                for b in range(_S):
                    pltpu.make_async_copy(tab_hbm.at[idx_bufs[p].at[b]],
                                          rows_v.at[b], gsem.at[b]).wait()
                    pltpu.async_copy(rows_v.at[b],
                                     out_hbm.at[pl.ds((row0 + b) * _G, _G)],
                                     osem.at[b])
                @pl.when(mm < _NOUT // 2 - 1)
                def _prefetch():
                    pltpu.async_copy(
                        idx_hbm.at[pl.ds(row0 + 2 * _S, _S)], idx_bufs[p],
                        isem.at[p])
            return 0

        lax.fori_loop(0, _NOUT // 2, outer, 0)

        last_row0 = base_row + (_NOUT - 1) * _S
        for b in range(_S):
            pltpu.make_async_copy(
                rows_v.at[b],
                out_hbm.at[pl.ds((last_row0 + b) * _G, _G)],
                osem.at[b]).wait()

    return k(idx2d, table)


def kernel(word_idx, emb0, emb1):
    del emb1
    idx2d = word_idx.reshape(_N_TOTAL // _G, _G)
    out = _gather(idx2d, emb0)
    return out.reshape(_B, _L, _D)
